# baseline (speedup 1.0000x reference)
import functools

import jax
import jax.numpy as jnp
from jax import lax
from jax.experimental import pallas as pl
from jax.experimental.pallas import tpu as pltpu
from jax.experimental.pallas import tpu_sc as plsc

_N = 7056
_PAD = 7168
_NW = 16
_CHUNK = _PAD // _NW
_VECS = _CHUNK // 16
_TAIL = _N - (_NW - 1) * _CHUNK


def _bfly(v, op):
    iota = lax.iota(jnp.int32, 16)
    for k in (8, 4, 2, 1):
        v = op(v, v.at[iota ^ k].get(mode="promise_in_bounds"))
    return v


_mesh = plsc.VectorSubcoreMesh(
    core_axis_name="c", subcore_axis_name="s", num_cores=2, num_subcores=16
)


@functools.partial(
    pl.kernel,
    out_type=jax.ShapeDtypeStruct((_N,), jnp.float32),
    mesh=_mesh,
    scratch_types=[
        pltpu.VMEM((_CHUNK,), jnp.float32),
        pltpu.VMEM((_CHUNK,), jnp.float32),
        pltpu.VMEM((16,), jnp.float32),
        pltpu.VMEM((16,), jnp.float32),
        pltpu.VMEM((16,), jnp.int32),
        pltpu.VMEM((32, 16), jnp.float32),
        pltpu.VMEM((16,), jnp.float32),
        pltpu.HBM((32, 16), jnp.float32),
    ],
)
def _sc_argmax_onehot(x_hbm, o_hbm, xbuf, zbuf, vbuf, fbuf, ibuf, msbuf,
                      ohbuf, sh):
    c = lax.axis_index("c")
    s = lax.axis_index("s")

    @pl.when(c == 0)
    def _core0():
        iota = lax.iota(jnp.int32, 16)
        base = s * _CHUNK
        pltpu.sync_copy(x_hbm.at[pl.ds(base, _CHUNK)], xbuf)

        zeros = jnp.zeros((16,), jnp.float32)
        for j in range(_VECS):
            zbuf[pl.ds(j * 16, 16)] = zeros

        m = xbuf[pl.ds(0, 16)]
        mi = iota + base
        for j in range(1, _VECS):
            v = xbuf[pl.ds(j * 16, 16)]
            upd = v > m
            m = jnp.where(upd, v, m)
            mi = jnp.where(upd, iota + (base + j * 16), mi)

        wmaxv = _bfly(m, jnp.maximum)
        widxv = _bfly(jnp.where(m == wmaxv, mi, _PAD), jnp.minimum)
        vbuf[...] = wmaxv
        fbuf[...] = widxv.astype(jnp.float32)
        pltpu.sync_copy(vbuf, sh.at[s])
        pltpu.sync_copy(fbuf, sh.at[s + 16])

        @pl.when(s < _NW - 1)
        def _zero_full():
            pltpu.sync_copy(zbuf, o_hbm.at[pl.ds(base, _CHUNK)])

        @pl.when(s == _NW - 1)
        def _zero_tail():
            pltpu.sync_copy(zbuf.at[pl.ds(0, _TAIL)],
                            o_hbm.at[pl.ds(base, _TAIL)])

        plsc.subcore_barrier()

        @pl.when(s == 0)
        def _merge():
            pltpu.sync_copy(sh, msbuf)
            vrows = [msbuf[r] for r in range(_NW)]
            irows = [msbuf[16 + r] for r in range(_NW)]
            gv = vrows[0]
            for r in range(1, _NW):
                gv = jnp.maximum(gv, vrows[r])
            givf = jnp.where(vrows[0] == gv, irows[0], float(_PAD))
            for r in range(1, _NW):
                givf = jnp.minimum(
                    givf, jnp.where(vrows[r] == gv, irows[r], float(_PAD)))
            giv = givf.astype(jnp.int32)
            ohbuf[...] = jnp.where(iota == (giv & 15), 1.0,
                                   0.0).astype(jnp.float32)
            win = (giv[0] // 16) * 16
            pltpu.sync_copy(ohbuf, o_hbm.at[pl.ds(win, 16)])


@jax.jit
def kernel(inputs):
    flat = jnp.reshape(inputs, (_N,))
    xpad = jnp.concatenate(
        [flat, jnp.full((_PAD - _N,), -jnp.inf, jnp.float32)])
    out = _sc_argmax_onehot(xpad)
    return jnp.reshape(out, (1, _N))


# speedup vs baseline: 1.0906x; 1.0906x over previous
import functools

import jax
import jax.numpy as jnp
from jax import lax
from jax.experimental import pallas as pl
from jax.experimental.pallas import tpu as pltpu
from jax.experimental.pallas import tpu_sc as plsc

_N = 7056
_PAD = 7168
_NW = 16
_CHUNK = _PAD // _NW
_VECS = _CHUNK // 16
_TAIL = _N - (_NW - 1) * _CHUNK


def _bfly(v, op):
    iota = lax.iota(jnp.int32, 16)
    for k in (8, 4, 2, 1):
        v = op(v, v.at[iota ^ k].get(mode="promise_in_bounds"))
    return v


_mesh = plsc.VectorSubcoreMesh(
    core_axis_name="c", subcore_axis_name="s", num_cores=1, num_subcores=16
)


@functools.partial(
    pl.kernel,
    out_type=jax.ShapeDtypeStruct((_N,), jnp.float32),
    mesh=_mesh,
    scratch_types=[
        pltpu.VMEM((_CHUNK,), jnp.float32),
        pltpu.VMEM((_CHUNK,), jnp.float32),
        pltpu.VMEM((16,), jnp.float32),
        pltpu.VMEM((16,), jnp.float32),
        pltpu.VMEM((16,), jnp.int32),
        pltpu.VMEM((32, 16), jnp.float32),
        pltpu.VMEM((16,), jnp.float32),
        pltpu.HBM((32, 16), jnp.float32),
    ],
)
def _sc_argmax_onehot(x_hbm, o_hbm, xbuf, zbuf, vbuf, fbuf, ibuf, msbuf,
                      ohbuf, sh):
    c = lax.axis_index("c")
    s = lax.axis_index("s")

    @pl.when(c == 0)
    def _core0():
        iota = lax.iota(jnp.int32, 16)
        base = s * _CHUNK
        pltpu.sync_copy(x_hbm.at[pl.ds(base, _CHUNK)], xbuf)

        zeros = jnp.zeros((16,), jnp.float32)
        for j in range(_VECS):
            zbuf[pl.ds(j * 16, 16)] = zeros

        m = xbuf[pl.ds(0, 16)]
        mi = iota + base
        for j in range(1, _VECS):
            v = xbuf[pl.ds(j * 16, 16)]
            upd = v > m
            m = jnp.where(upd, v, m)
            mi = jnp.where(upd, iota + (base + j * 16), mi)

        wmaxv = _bfly(m, jnp.maximum)
        widxv = _bfly(jnp.where(m == wmaxv, mi, _PAD), jnp.minimum)
        vbuf[...] = wmaxv
        fbuf[...] = widxv.astype(jnp.float32)
        pltpu.sync_copy(vbuf, sh.at[s])
        pltpu.sync_copy(fbuf, sh.at[s + 16])

        @pl.when(s < _NW - 1)
        def _zero_full():
            pltpu.sync_copy(zbuf, o_hbm.at[pl.ds(base, _CHUNK)])

        @pl.when(s == _NW - 1)
        def _zero_tail():
            pltpu.sync_copy(zbuf.at[pl.ds(0, _TAIL)],
                            o_hbm.at[pl.ds(base, _TAIL)])

        plsc.subcore_barrier()

        @pl.when(s == 0)
        def _merge():
            pltpu.sync_copy(sh, msbuf)
            vrows = [msbuf[r] for r in range(_NW)]
            irows = [msbuf[16 + r] for r in range(_NW)]
            gv = vrows[0]
            for r in range(1, _NW):
                gv = jnp.maximum(gv, vrows[r])
            givf = jnp.where(vrows[0] == gv, irows[0], float(_PAD))
            for r in range(1, _NW):
                givf = jnp.minimum(
                    givf, jnp.where(vrows[r] == gv, irows[r], float(_PAD)))
            giv = givf.astype(jnp.int32)
            ohbuf[...] = jnp.where(iota == (giv & 15), 1.0,
                                   0.0).astype(jnp.float32)
            win = (giv[0] // 16) * 16
            pltpu.sync_copy(ohbuf, o_hbm.at[pl.ds(win, 16)])


@jax.jit
def kernel(inputs):
    flat = jnp.reshape(inputs, (_N,))
    xpad = jnp.concatenate(
        [flat, jnp.full((_PAD - _N,), -jnp.inf, jnp.float32)])
    out = _sc_argmax_onehot(xpad)
    return jnp.reshape(out, (1, _N))
